# Initial kernel scaffold; baseline (speedup 1.0000x reference)
import jax
import jax.numpy as jnp
from jax.experimental import pallas as pl


def kernel(x_in, W1a, b1a, W1b, b1b, W2a, b2a, W2b, b2b, Wout, bout):
    raise NotImplementedError("write your pallas kernel here")


# speedup vs baseline: 6.2199x; 6.2199x over previous
import functools
import jax
import jax.numpy as jnp
from jax.experimental import pallas as pl
from jax.experimental.pallas import tpu as pltpu

_RADIUS = 0.1


def _gelu(v):
    return 0.5 * v * (1.0 + jax.lax.erf(v * 0.7071067811865476))


def _conv_body(xT_ref, fT_ref, xbT_ref, fb_ref, WaTt_ref, Wab_ref, ba_ref,
               Wb_ref, bb_ref, out_ref, *, n_chunks, bj):
    r2 = _RADIUS * _RADIUS
    Q = jnp.dot(fb_ref[...], Wab_ref[...],
                preferred_element_type=jnp.float32) + ba_ref[...]
    xbT = xbT_ref[...]
    bi = xbT.shape[1]
    w = Q.shape[1]

    def chunk(c, carry):
        S, cnt = carry
        xj = xT_ref[:, pl.ds(c * bj, bj)]
        fj = fT_ref[:, pl.ds(c * bj, bj)]
        PjT = jnp.dot(WaTt_ref[...], fj,
                      preferred_element_type=jnp.float32)
        diff = xbT[:, :, None] - xj[:, None, :]
        d2 = jnp.sum(diff * diff, axis=0)
        m = d2 < r2
        H = _gelu(Q[:, :, None] + PjT[None, :, :])
        Hm = jnp.where(m[:, None, :], H, 0.0)
        S = S + jnp.sum(Hm, axis=2)
        cnt = cnt + jnp.sum(m.astype(jnp.float32), axis=1, keepdims=True)
        return S, cnt

    S0 = jnp.zeros((bi, w), jnp.float32)
    c0 = jnp.zeros((bi, 1), jnp.float32)
    S, cnt = jax.lax.fori_loop(0, n_chunks, chunk, (S0, c0))
    mean = S / jnp.maximum(cnt, 1.0)
    out_ref[...] = jnp.dot(mean, Wb_ref[...],
                           preferred_element_type=jnp.float32) + bb_ref[...]


def _conv(x, feat, Wa, ba, Wb, bb, bi=256, bj=256):
    n = x.shape[0]
    cf = feat.shape[1]
    w = Wa.shape[1]
    xT = x.T
    fT = feat.T
    WaTt = Wa[:cf].T
    Wab = Wa[cf:]
    ba2 = ba.reshape(1, w)
    bb2 = bb.reshape(1, w)
    grid = n // bi
    body = functools.partial(_conv_body, n_chunks=n // bj, bj=bj)
    return pl.pallas_call(
        body,
        grid=(grid,),
        in_specs=[
            pl.BlockSpec((3, n), lambda i: (0, 0)),
            pl.BlockSpec((cf, n), lambda i: (0, 0)),
            pl.BlockSpec((3, bi), lambda i: (0, i)),
            pl.BlockSpec((bi, cf), lambda i: (i, 0)),
            pl.BlockSpec((w, cf), lambda i: (0, 0)),
            pl.BlockSpec((cf, w), lambda i: (0, 0)),
            pl.BlockSpec((1, w), lambda i: (0, 0)),
            pl.BlockSpec((w, w), lambda i: (0, 0)),
            pl.BlockSpec((1, w), lambda i: (0, 0)),
        ],
        out_specs=pl.BlockSpec((bi, w), lambda i: (i, 0)),
        out_shape=jax.ShapeDtypeStruct((n, w), jnp.float32),
    )(xT, fT, xT, feat, WaTt, Wab, ba2, Wb, bb2)


def _head_body(h_ref, wT_ref, b_ref, out_ref):
    g = _gelu(h_ref[...])
    out_ref[...] = (jnp.sum(g * wT_ref[...], axis=1, keepdims=True)
                    + b_ref[...])


def _head(h, Wout, bout):
    n, w = h.shape
    return pl.pallas_call(
        _head_body,
        in_specs=[
            pl.BlockSpec((n, w), lambda: (0, 0)),
            pl.BlockSpec((1, w), lambda: (0, 0)),
            pl.BlockSpec((1, 1), lambda: (0, 0)),
        ],
        out_specs=pl.BlockSpec((n, 1), lambda: (0, 0)),
        out_shape=jax.ShapeDtypeStruct((n, 1), jnp.float32),
    )(h, Wout.T, bout.reshape(1, 1))


def kernel(x_in, W1a, b1a, W1b, b1b, W2a, b2a, W2b, b2b, Wout, bout):
    x = jnp.squeeze(x_in)
    n = x.shape[0]
    npad = -n % 256
    xp = jnp.concatenate(
        [x, jnp.full((npad, x.shape[1]), 100.0, jnp.float32)], axis=0)
    h1 = _conv(xp, xp, W1a, b1a, W1b, b1b)
    h2 = _conv(xp, h1, W2a, b2a, W2b, b2b)
    return _head(h2[:n], Wout, bout)


# speedup vs baseline: 25.9554x; 4.1729x over previous
import functools
import jax
import jax.numpy as jnp
from jax.experimental import pallas as pl
from jax.experimental.pallas import tpu as pltpu

_RADIUS = 0.1


def _gelu(v):
    return 0.5 * v * (1.0 + jax.lax.erf(v * 0.7071067811865476))


def _conv_body(xT_ref, fT_ref, zlo_ref, zhi_ref, xbT_ref, fb_ref, WaTt_ref,
               Wab_ref, ba_ref, Wb_ref, bb_ref, out_ref, *, n_chunks, bj):
    r2 = _RADIUS * _RADIUS
    Q = jnp.dot(fb_ref[...], Wab_ref[...],
                preferred_element_type=jnp.float32) + ba_ref[...]
    xbT = xbT_ref[...]
    bi = xbT.shape[1]
    w = Q.shape[1]

    zb = xbT[2, :]
    zb_lo = jnp.min(zb)
    zb_hi = jnp.max(zb)
    c_start = jnp.sum((zhi_ref[...] < zb_lo - _RADIUS).astype(jnp.int32))
    c_end = n_chunks - jnp.sum(
        (zlo_ref[...] > zb_hi + _RADIUS).astype(jnp.int32))

    def chunk(c, carry):
        S, cnt = carry
        xj = xT_ref[:, pl.ds(c * bj, bj)]
        fj = fT_ref[:, pl.ds(c * bj, bj)]
        PjT = jnp.dot(WaTt_ref[...], fj,
                      preferred_element_type=jnp.float32)
        diff = xbT[:, :, None] - xj[:, None, :]
        d2 = jnp.sum(diff * diff, axis=0)
        m = d2 < r2
        H = _gelu(Q[:, :, None] + PjT[None, :, :])
        Hm = jnp.where(m[:, None, :], H, 0.0)
        S = S + jnp.sum(Hm, axis=2)
        cnt = cnt + jnp.sum(m.astype(jnp.float32), axis=1, keepdims=True)
        return S, cnt

    S0 = jnp.zeros((bi, w), jnp.float32)
    c0 = jnp.zeros((bi, 1), jnp.float32)
    S, cnt = jax.lax.fori_loop(c_start, c_end, chunk, (S0, c0))
    mean = S / jnp.maximum(cnt, 1.0)
    out_ref[...] = jnp.dot(mean, Wb_ref[...],
                           preferred_element_type=jnp.float32) + bb_ref[...]


def _conv(x, feat, zlo, zhi, Wa, ba, Wb, bb, bi=256, bj=256):
    n = x.shape[0]
    cf = feat.shape[1]
    w = Wa.shape[1]
    nc = n // bj
    xT = x.T
    fT = feat.T
    WaTt = Wa[:cf].T
    Wab = Wa[cf:]
    ba2 = ba.reshape(1, w)
    bb2 = bb.reshape(1, w)
    grid = n // bi
    body = functools.partial(_conv_body, n_chunks=nc, bj=bj)
    return pl.pallas_call(
        body,
        grid=(grid,),
        in_specs=[
            pl.BlockSpec((3, n), lambda i: (0, 0)),
            pl.BlockSpec((cf, n), lambda i: (0, 0)),
            pl.BlockSpec((1, nc), lambda i: (0, 0)),
            pl.BlockSpec((1, nc), lambda i: (0, 0)),
            pl.BlockSpec((3, bi), lambda i: (0, i)),
            pl.BlockSpec((bi, cf), lambda i: (i, 0)),
            pl.BlockSpec((w, cf), lambda i: (0, 0)),
            pl.BlockSpec((cf, w), lambda i: (0, 0)),
            pl.BlockSpec((1, w), lambda i: (0, 0)),
            pl.BlockSpec((w, w), lambda i: (0, 0)),
            pl.BlockSpec((1, w), lambda i: (0, 0)),
        ],
        out_specs=pl.BlockSpec((bi, w), lambda i: (i, 0)),
        out_shape=jax.ShapeDtypeStruct((n, w), jnp.float32),
    )(xT, fT, zlo, zhi, xT, feat, WaTt, Wab, ba2, Wb, bb2)


def _head_body(h_ref, wT_ref, b_ref, out_ref):
    g = _gelu(h_ref[...])
    out_ref[...] = (jnp.sum(g * wT_ref[...], axis=1, keepdims=True)
                    + b_ref[...])


def _head(h, Wout, bout):
    n, w = h.shape
    return pl.pallas_call(
        _head_body,
        in_specs=[
            pl.BlockSpec((n, w), lambda: (0, 0)),
            pl.BlockSpec((1, w), lambda: (0, 0)),
            pl.BlockSpec((1, 1), lambda: (0, 0)),
        ],
        out_specs=pl.BlockSpec((n, 1), lambda: (0, 0)),
        out_shape=jax.ShapeDtypeStruct((n, 1), jnp.float32),
    )(h, Wout.T, bout.reshape(1, 1))


def kernel(x_in, W1a, b1a, W1b, b1b, W2a, b2a, W2b, b2b, Wout, bout):
    x = jnp.squeeze(x_in)
    n = x.shape[0]
    bj = 256
    perm = jnp.argsort(x[:, 2])
    xs = x[perm]
    npad = -n % 256
    xp = jnp.concatenate(
        [xs, jnp.full((npad, x.shape[1]), 100.0, jnp.float32)], axis=0)
    zcol = xp[:, 2]
    zlo = zcol[0::bj].reshape(1, -1)
    zhi = zcol[bj - 1::bj].reshape(1, -1)
    h1 = _conv(xp, xp, zlo, zhi, W1a, b1a, W1b, b1b)
    h2 = _conv(xp, h1, zlo, zhi, W2a, b2a, W2b, b2b)
    res = _head(h2[:n], Wout, bout)
    return jnp.zeros((n, 1), jnp.float32).at[perm].set(res)
